# baseline (speedup 1.0000x reference)
import functools
import math

import jax
import jax.numpy as jnp
from jax import lax
from jax.experimental import pallas as pl
from jax.experimental.pallas import tpu as pltpu
from jax.experimental.pallas import tpu_sc as plsc

_B = 16
_L = 2048
_T = _B * _L
_D = 32
_K = 16
_LOG2PI = math.log(2.0 * math.pi)
_LN2 = math.log(2.0)
_RPW = _L // 8


_GB = 8
_BC = _B // _GB
_TC_ = _BC * _L


def _emit_body(x_ref, logA_ref, logpi_ref, means_ref, logvars_ref,
               eb_ref, msum_ref):
    x = x_ref[:]
    lv = logvars_ref[:]
    iv = jnp.exp(-lv)
    w = means_ref[:] * iv
    cst = jnp.sum(means_ref[:] * w + lv, axis=1, keepdims=True) + _D * _LOG2PI
    dn = (((1,), (1,)), ((), ()))
    q = (jax.lax.dot_general(x * x, iv, dn, preferred_element_type=jnp.float32)
         - 2.0 * jax.lax.dot_general(x, w, dn, preferred_element_type=jnp.float32))
    logb = (-0.5 * (q + cst.reshape(1, _K))).reshape(_BC, _L, _K)
    m3 = jnp.max(logb, axis=2, keepdims=True)
    eb4 = jnp.exp(logb - m3).reshape(_BC, _L // 8, 8, _K)
    eb_ref[:] = jnp.concatenate(
        [eb4[:, :, a, :].reshape(_BC * (_L // 8), _K) for a in range(8)],
        axis=1)
    msum_ref[:] = jnp.sum(m3, axis=1).reshape(1, _BC, 1)


_GDN = lax.GatherDimensionNumbers(offset_dims=(), collapsed_slice_dims=(0,),
                                  start_index_map=(0,))


def _lanes(a, idx):
    return lax.gather(a, idx[:, None], _GDN, (1,),
                      mode=lax.GatherScatterMode.PROMISE_IN_BOUNDS)


def _scan_body(eb_hbm, aux_hbm, alpha_out, e_out, ebv, auxv, aov, eov):
    wid = lax.axis_index("s")

    @pl.when(lax.axis_index("c") == 0)
    def _():
        pltpu.sync_copy(eb_hbm.at[pl.ds(wid * _RPW, _RPW)], ebv)
        pltpu.sync_copy(aux_hbm, auxv)
        pi = jnp.exp(auxv[0, 0:_K])
        arows = [jnp.exp(auxv[(_K + _K * i) // 128,
                              pl.ds(((_K + _K * i) % 128), _K)])
                 for i in range(_K)]
        bidx = [jnp.full((_K,), i, jnp.int32) for i in range(_K)]
        lane = lax.iota(jnp.int32, _K)
        fly = [lane ^ d for d in (8, 4, 2, 1)]

        def matvec(a):
            terms = [arows[i] * _lanes(a, bidx[i]) for i in range(_K)]
            while len(terms) > 1:
                terms = [terms[2 * i] + terms[2 * i + 1]
                         for i in range(len(terms) // 2)]
            return terms[0]

        def renorm(a, ev):
            cv = a
            for f in fly:
                cv = cv + _lanes(cv, f)
            fac = jnp.full((_K,), 1.0, jnp.float32)
            boost = jnp.zeros((_K,), jnp.float32)
            for k in (64, 32, 16, 8, 4, 2, 1):
                cond = (cv * fac) < (2.0 ** (-k))
                fac = jnp.where(cond, fac * (2.0 ** k), fac)
                boost = jnp.where(cond, boost + float(k), boost)
            return a * fac, ev - boost

        zero = jnp.zeros((_K,), jnp.float32)
        alpha, ev = renorm(pi * ebv[0, 0:_K], zero)
        for t in range(1, 8):
            alpha = matvec(alpha) * ebv[0, pl.ds(t * _K, _K)]
            if t in (4, 7):
                alpha, ev = renorm(alpha, ev)

        def row_block(r, carry):
            alpha, ev = carry
            for j in range(8):
                alpha = matvec(alpha) * ebv[r, pl.ds(j * _K, _K)]
                if j in (3, 7):
                    alpha, ev = renorm(alpha, ev)
            return alpha, ev

        alpha, ev = lax.fori_loop(1, _RPW, row_block, (alpha, ev))
        for i in range(8):
            aov[pl.ds(i * _K, _K)] = alpha if i == 0 else zero
            eov[pl.ds(i * _K, _K)] = ev if i == 0 else zero
        pltpu.sync_copy(aov, alpha_out.at[wid])
        pltpu.sync_copy(eov, e_out.at[wid])


@functools.partial(jax.jit, static_argnames=())
def kernel(X, log_A, log_pi, means, log_vars):
    eb, msum = pl.pallas_call(
        _emit_body,
        grid=(_GB,),
        in_specs=[
            pl.BlockSpec((_TC_, _D), lambda i: (i, 0)),
            pl.BlockSpec((_K, _K), lambda i: (0, 0)),
            pl.BlockSpec((1, _K), lambda i: (0, 0)),
            pl.BlockSpec((_K, _D), lambda i: (0, 0)),
            pl.BlockSpec((_K, _D), lambda i: (0, 0)),
        ],
        out_specs=[
            pl.BlockSpec((_BC * _RPW, 8 * _K), lambda i: (i, 0)),
            pl.BlockSpec((1, _BC, 1), lambda i: (i, 0, 0)),
        ],
        out_shape=[
            jax.ShapeDtypeStruct((_B * _RPW, 8 * _K), jnp.float32),
            jax.ShapeDtypeStruct((_GB, _BC, 1), jnp.float32),
        ],
    )(X, log_A, log_pi.reshape(1, _K), means, log_vars)

    aux = jnp.concatenate(
        [log_pi, log_A.reshape(_K * _K), jnp.zeros((112,), jnp.float32)]
    ).reshape(3, 128)

    mesh = plsc.VectorSubcoreMesh(core_axis_name="c", subcore_axis_name="s")
    alpha_rows, e_rows = pl.kernel(
        _scan_body,
        out_type=[
            jax.ShapeDtypeStruct((_B, 128), jnp.float32),
            jax.ShapeDtypeStruct((_B, 128), jnp.float32),
        ],
        mesh=mesh,
        scratch_types=[
            pltpu.VMEM((_RPW, 8 * _K), jnp.float32),
            pltpu.VMEM((3, 128), jnp.float32),
            pltpu.VMEM((128,), jnp.float32),
            pltpu.VMEM((128,), jnp.float32),
        ],
    )(eb, aux)

    return (msum.reshape(_B) + _LN2 * e_rows[:, 0]
            + jnp.log(jnp.sum(alpha_rows[:, 0:_K], axis=1)))
